# baseline (speedup 1.0000x reference)
import functools

import jax
import jax.numpy as jnp
from jax.experimental import pallas as pl
from jax.experimental.pallas import tpu as pltpu


def _body(x_any, xh_any, w1t_ref, w2t_ref, wg_ref, bg_ref, o_any, ys_ref,
          pat_ref, mask_ref, hpat_ref, *, b, c, h, w, hb, inv_n):
    nh = h // hb
    m = hb * w

    ri = jax.lax.broadcasted_iota(jnp.int32, (1, hb, w), 1)
    ci = jax.lax.broadcasted_iota(jnp.int32, (1, hb, w), 2)
    jcol = (ci > 0).astype(jnp.float32)
    pat_ref[0] = jax.lax.rsqrt(
        1.0 + (ri > 0).astype(jnp.float32) + jcol).reshape(1, m)
    pat_ref[1] = jax.lax.rsqrt(2.0 + jcol).reshape(1, m)
    mask_ref[...] = jcol.reshape(1, m)
    cj = jax.lax.broadcasted_iota(jnp.int32, (1, w), 1)
    hpat_ref[...] = jax.lax.rsqrt(2.0 + (cj > 0).astype(jnp.float32))

    def se_body(x_blk):
        bi = pl.program_id(0)
        hi = pl.program_id(1)
        part = jnp.sum(x_blk[0], axis=(1, 2))[None, :]

        @pl.when(hi == 0)
        def _():
            ys_ref[bi] = part

        @pl.when(hi > 0)
        def _():
            ys_ref[bi] += part

    pltpu.emit_pipeline(
        se_body,
        grid=(b, nh),
        in_specs=[pl.BlockSpec((1, c, hb, w), lambda bi, hi: (bi, 0, hi, 0))],
    )(x_any)

    mean = ys_ref[...].reshape(b, c) * inv_n
    t = jnp.maximum(
        jnp.dot(mean, w1t_ref[...], preferred_element_type=jnp.float32), 0.0)
    yv = jax.nn.sigmoid(
        jnp.dot(t, w2t_ref[...], preferred_element_type=jnp.float32))
    ys_ref[...] = yv.reshape(b, 1, c)

    def gcn_body(x_blk, xh_blk, o_blk):
        bi = pl.program_id(0)
        hi = pl.program_id(1)
        y = ys_ref[bi, 0]
        wgy = wg_ref[...] * y[None, :]
        zw = jnp.dot(wgy, x_blk[0].reshape(c, m),
                     preferred_element_type=jnp.float32)

        dinv = pat_ref[jnp.minimum(hi, 1)]
        z = zw * dinv

        zh = jnp.dot(wgy, xh_blk[0, 0], preferred_element_type=jnp.float32)
        zh = jnp.where(hi == 0, 0.0, zh * hpat_ref[...])

        zd = jnp.concatenate([zh, z[:, :m - w]], axis=1)
        zr = jnp.roll(z, 1, axis=1) * mask_ref[...]

        res = (z + zd + zr) * dinv + bg_ref[0][:, None]
        o_blk[0] = res.reshape(c, hb, w)

    pltpu.emit_pipeline(
        gcn_body,
        grid=(b, nh),
        in_specs=[
            pl.BlockSpec((1, c, hb, w), lambda bi, hi: (bi, 0, hi, 0)),
            pl.BlockSpec((1, 1, c, w), lambda bi, hi: (bi, hi, 0, 0)),
        ],
        out_specs=[pl.BlockSpec((1, c, hb, w), lambda bi, hi: (bi, 0, hi, 0))],
    )(x_any, xh_any, o_any)


@jax.jit
def kernel(x, W1, W2, Wg, bg):
    b, c, h, w = x.shape
    hb = 64
    n = h * w

    halo = jnp.concatenate(
        [jnp.zeros((b, c, 1, w), x.dtype), x[:, :, hb - 1:h - 1:hb, :]],
        axis=2).transpose(0, 2, 1, 3)

    out = pl.pallas_call(
        functools.partial(_body, b=b, c=c, h=h, w=w, hb=hb, inv_n=1.0 / n),
        in_specs=[
            pl.BlockSpec(memory_space=pl.ANY),
            pl.BlockSpec(memory_space=pl.ANY),
            pl.BlockSpec(memory_space=pltpu.VMEM),
            pl.BlockSpec(memory_space=pltpu.VMEM),
            pl.BlockSpec(memory_space=pltpu.VMEM),
            pl.BlockSpec(memory_space=pltpu.VMEM),
        ],
        out_specs=pl.BlockSpec(memory_space=pl.ANY),
        out_shape=jax.ShapeDtypeStruct((b, c, h, w), jnp.float32),
        scratch_shapes=[
            pltpu.VMEM((b, 1, c), jnp.float32),
            pltpu.VMEM((2, 1, hb * w), jnp.float32),
            pltpu.VMEM((1, hb * w), jnp.float32),
            pltpu.VMEM((1, w), jnp.float32),
        ],
        compiler_params=pltpu.CompilerParams(vmem_limit_bytes=67108864),
    )(x, halo, W1.T, W2.T, Wg, bg[None, :])
    return out


# speedup vs baseline: 1.0157x; 1.0157x over previous
import functools

import jax
import jax.numpy as jnp
from jax.experimental import pallas as pl
from jax.experimental.pallas import tpu as pltpu


def _body(x_any, xh_any, w1t_ref, w2t_ref, wg_ref, bg_ref, o_any, ys_ref, *,
          b, c, h, w, hb, inv_n):
    nh = h // hb
    m = hb * w

    def se_body(x_blk):
        bi = pl.program_id(0)
        hi = pl.program_id(1)
        part = jnp.sum(x_blk[0], axis=(1, 2))[None, :]

        @pl.when(hi == 0)
        def _():
            ys_ref[bi] = part

        @pl.when(hi > 0)
        def _():
            ys_ref[bi] += part

    pltpu.emit_pipeline(
        se_body,
        grid=(b, nh),
        in_specs=[pl.BlockSpec((1, c, hb, w), lambda bi, hi: (bi, 0, hi, 0))],
    )(x_any)

    mean = ys_ref[...].reshape(b, c) * inv_n
    t = jnp.maximum(
        jnp.dot(mean, w1t_ref[...], preferred_element_type=jnp.float32), 0.0)
    yv = jax.nn.sigmoid(
        jnp.dot(t, w2t_ref[...], preferred_element_type=jnp.float32))
    ys_ref[...] = yv.reshape(b, 1, c)

    def gcn_body(x_blk, xh_blk, o_blk):
        bi = pl.program_id(0)
        hi = pl.program_id(1)
        y = ys_ref[bi, 0]
        wgy = wg_ref[...] * y[None, :]
        zw = jnp.dot(wgy, x_blk[0].reshape(c, m),
                     preferred_element_type=jnp.float32)

        ri = jax.lax.broadcasted_iota(jnp.int32, (1, hb, w), 1) + hi * hb
        ci = jax.lax.broadcasted_iota(jnp.int32, (1, hb, w), 2)
        dinv = jax.lax.rsqrt(1.0 + (ri > 0).astype(jnp.float32)
                             + (ci > 0).astype(jnp.float32)).reshape(1, m)
        maskj = (ci > 0).astype(jnp.float32).reshape(1, m)
        z = zw * dinv

        zh = jnp.dot(wgy, xh_blk[0, 0], preferred_element_type=jnp.float32)
        cj = jax.lax.broadcasted_iota(jnp.int32, (1, w), 1)
        dinv_h = jax.lax.rsqrt(2.0 + (cj > 0).astype(jnp.float32))
        zh = jnp.where(hi == 0, 0.0, zh * dinv_h)

        zd = jnp.concatenate([zh, z[:, :m - w]], axis=1)
        zr = jnp.roll(z, 1, axis=1) * maskj

        res = (z + zd + zr) * dinv + bg_ref[0][:, None]
        o_blk[0] = res.reshape(c, hb, w)

    pltpu.emit_pipeline(
        gcn_body,
        grid=(b, nh),
        in_specs=[
            pl.BlockSpec((1, c, hb, w), lambda bi, hi: (bi, 0, hi, 0)),
            pl.BlockSpec((1, 1, c, w), lambda bi, hi: (bi, hi, 0, 0)),
        ],
        out_specs=[pl.BlockSpec((1, c, hb, w), lambda bi, hi: (bi, 0, hi, 0))],
    )(x_any, xh_any, o_any)


@jax.jit
def kernel(x, W1, W2, Wg, bg):
    b, c, h, w = x.shape
    hb = 64
    n = h * w

    halo = jnp.concatenate(
        [jnp.zeros((b, c, 1, w), x.dtype), x[:, :, hb - 1:h - 1:hb, :]],
        axis=2).transpose(0, 2, 1, 3)

    out = pl.pallas_call(
        functools.partial(_body, b=b, c=c, h=h, w=w, hb=hb, inv_n=1.0 / n),
        in_specs=[
            pl.BlockSpec(memory_space=pl.ANY),
            pl.BlockSpec(memory_space=pl.ANY),
            pl.BlockSpec(memory_space=pltpu.VMEM),
            pl.BlockSpec(memory_space=pltpu.VMEM),
            pl.BlockSpec(memory_space=pltpu.VMEM),
            pl.BlockSpec(memory_space=pltpu.VMEM),
        ],
        out_specs=pl.BlockSpec(memory_space=pl.ANY),
        out_shape=jax.ShapeDtypeStruct((b, c, h, w), jnp.float32),
        scratch_shapes=[pltpu.VMEM((b, 1, c), jnp.float32)],
        compiler_params=pltpu.CompilerParams(vmem_limit_bytes=67108864),
    )(x, halo, W1.T, W2.T, Wg, bg[None, :])
    return out


# speedup vs baseline: 1.0166x; 1.0008x over previous
import functools

import jax
import jax.numpy as jnp
from jax.experimental import pallas as pl
from jax.experimental.pallas import tpu as pltpu


def _body(x_any, xh_any, w1t_ref, w2t_ref, wg_ref, bg_ref, o_any, ys_ref, *,
          b, c, h, w, hb, inv_n):
    nh = h // hb
    m = hb * w

    def se_body(x_blk):
        bi = pl.program_id(0)
        hi = pl.program_id(1)
        part = jnp.sum(x_blk[0], axis=(1, 2))[None, :]

        @pl.when(hi == 0)
        def _():
            ys_ref[bi] = part

        @pl.when(hi > 0)
        def _():
            ys_ref[bi] += part

    pltpu.emit_pipeline(
        se_body,
        grid=(b, nh),
        in_specs=[pl.BlockSpec((1, c, hb, w), lambda bi, hi: (bi, 0, hi, 0))],
        dimension_semantics=(pltpu.PARALLEL, pltpu.PARALLEL),
    )(x_any)

    mean = ys_ref[...].reshape(b, c) * inv_n
    t = jnp.maximum(
        jnp.dot(mean, w1t_ref[...], preferred_element_type=jnp.float32), 0.0)
    yv = jax.nn.sigmoid(
        jnp.dot(t, w2t_ref[...], preferred_element_type=jnp.float32))
    ys_ref[...] = yv.reshape(b, 1, c)

    def gcn_body(x_blk, xh_blk, o_blk):
        bi = pl.program_id(0)
        hi = pl.program_id(1)
        y = ys_ref[bi, 0]
        wgy = wg_ref[...] * y[None, :]
        zw = jnp.dot(wgy, x_blk[0].reshape(c, m),
                     preferred_element_type=jnp.float32)

        ri = jax.lax.broadcasted_iota(jnp.int32, (1, hb, w), 1) + hi * hb
        ci = jax.lax.broadcasted_iota(jnp.int32, (1, hb, w), 2)
        dinv = jax.lax.rsqrt(1.0 + (ri > 0).astype(jnp.float32)
                             + (ci > 0).astype(jnp.float32)).reshape(1, m)
        maskj = (ci > 0).astype(jnp.float32).reshape(1, m)
        z = zw * dinv

        zh = jnp.dot(wgy, xh_blk[0, 0], preferred_element_type=jnp.float32)
        cj = jax.lax.broadcasted_iota(jnp.int32, (1, w), 1)
        dinv_h = jax.lax.rsqrt(2.0 + (cj > 0).astype(jnp.float32))
        zh = jnp.where(hi == 0, 0.0, zh * dinv_h)

        zd = jnp.concatenate([zh, z[:, :m - w]], axis=1)
        zr = jnp.roll(z, 1, axis=1) * maskj

        res = (z + zd + zr) * dinv + bg_ref[0][:, None]
        o_blk[0] = res.reshape(c, hb, w)

    pltpu.emit_pipeline(
        gcn_body,
        grid=(b, nh),
        in_specs=[
            pl.BlockSpec((1, c, hb, w), lambda bi, hi: (bi, 0, hi, 0)),
            pl.BlockSpec((1, 1, c, w), lambda bi, hi: (bi, hi, 0, 0)),
        ],
        out_specs=[pl.BlockSpec((1, c, hb, w), lambda bi, hi: (bi, 0, hi, 0))],
        dimension_semantics=(pltpu.PARALLEL, pltpu.PARALLEL),
    )(x_any, xh_any, o_any)


@jax.jit
def kernel(x, W1, W2, Wg, bg):
    b, c, h, w = x.shape
    hb = 64
    n = h * w

    halo = jnp.concatenate(
        [jnp.zeros((b, c, 1, w), x.dtype), x[:, :, hb - 1:h - 1:hb, :]],
        axis=2).transpose(0, 2, 1, 3)

    out = pl.pallas_call(
        functools.partial(_body, b=b, c=c, h=h, w=w, hb=hb, inv_n=1.0 / n),
        in_specs=[
            pl.BlockSpec(memory_space=pl.ANY),
            pl.BlockSpec(memory_space=pl.ANY),
            pl.BlockSpec(memory_space=pltpu.VMEM),
            pl.BlockSpec(memory_space=pltpu.VMEM),
            pl.BlockSpec(memory_space=pltpu.VMEM),
            pl.BlockSpec(memory_space=pltpu.VMEM),
        ],
        out_specs=pl.BlockSpec(memory_space=pl.ANY),
        out_shape=jax.ShapeDtypeStruct((b, c, h, w), jnp.float32),
        scratch_shapes=[pltpu.VMEM((b, 1, c), jnp.float32)],
        compiler_params=pltpu.CompilerParams(vmem_limit_bytes=67108864),
    )(x, halo, W1.T, W2.T, Wg, bg[None, :])
    return out
